# Initial kernel scaffold; baseline (speedup 1.0000x reference)
import jax
import jax.numpy as jnp
from jax.experimental import pallas as pl


def kernel(x, edge_index, W1, b1, W2, b2):
    raise NotImplementedError("write your pallas kernel here")


# speedup vs baseline: 3.0007x; 3.0007x over previous
import functools

import jax
import jax.numpy as jnp
from jax import lax
from jax.experimental import pallas as pl
from jax.experimental.pallas import tpu as pltpu
from jax.experimental.pallas import tpu_sc as plsc

NODE_DIM = 128
HIDDEN = 128
N_NODES = 10000
N_EDGES = 320000

NC = 2
NS = 16
NW = NC * NS
L = 16

EDGES_PER_W = N_EDGES // NW
CHUNK = 80
GROUPS = CHUNK // L
N_CHUNKS = EDGES_PER_W // CHUNK
NVJ = HIDDEN // L


def _z_body(x_ref, w1_ref, b1_ref, z_ref):
    z_ref[...] = (
        jnp.dot(x_ref[...], w1_ref[...], preferred_element_type=jnp.float32)
        + b1_ref[...]
    ) * 0.5


def _compute_z(x, W1, b1):
    return pl.pallas_call(
        _z_body,
        out_shape=jax.ShapeDtypeStruct((N_NODES, HIDDEN), jnp.float32),
    )(x, W1, b1.reshape(1, HIDDEN))


_BITREV = (0, 8, 4, 12, 2, 10, 6, 14, 1, 9, 5, 13, 3, 11, 7, 15)


_GDN = lax.GatherDimensionNumbers(
    offset_dims=(), collapsed_slice_dims=(0,), start_index_map=(0,))


def _lane_perm(a, idx):
    return lax.gather(a, idx[:, None], _GDN, (1,),
                      mode=lax.GatherScatterMode.PROMISE_IN_BOUNDS)


def _hsum16(vregs, lane):
    cur = list(vregs)
    for w in (16, 8, 4, 2):
        swap = lane ^ (w // 2)
        low = (lane % w) < (w // 2)
        nxt = []
        for p in range(0, len(cur), 2):
            a, b = cur[p], cur[p + 1]
            pa = _lane_perm(a, swap)
            pb = _lane_perm(b, swap)
            nxt.append(jnp.where(low, a, pb) + jnp.where(low, pa, b))
        cur = nxt
    return cur[0]


def _sc_body(z_hbm, ridx_hbm, cidx_hbm, w2_hbm, b2_hbm, out_hbm,
             ridx_v, cidx_v, rrows, crows, w2_v, b2_v, obuf, sem):
    wid = lax.axis_index("s") * NC + lax.axis_index("c")
    base_w = wid * EDGES_PER_W

    pltpu.sync_copy(w2_hbm, w2_v)
    pltpu.sync_copy(b2_hbm, b2_v)
    w2r = [w2_v[pl.ds(j * L, L)] for j in range(NVJ)]
    b2r = b2_v[...]
    lane = lax.iota(jnp.int32, L)

    def chunk_body(k, carry):
        base = base_w + k * CHUNK
        pltpu.sync_copy(ridx_hbm.at[pl.ds(base, CHUNK)], ridx_v)
        pltpu.sync_copy(cidx_hbm.at[pl.ds(base, CHUNK)], cidx_v)
        pltpu.async_copy(z_hbm.at[ridx_v], rrows, sem).wait()
        pltpu.async_copy(z_hbm.at[cidx_v], crows, sem).wait()

        def group_body(g, gcarry):
            gbase = g * L
            accs = []
            for i in range(L):
                e = gbase + _BITREV[i]
                acc = jnp.zeros((L,), jnp.float32)
                for j in range(NVJ):
                    rv = rrows[e, pl.ds(j * L, L)]
                    cv = crows[e, pl.ds(j * L, L)]
                    acc = acc + jnp.maximum(rv + cv, 0.0) * w2r[j]
                accs.append(acc)
            obuf[pl.ds(gbase, L)] = _hsum16(accs, lane) + b2r
            return gcarry

        lax.fori_loop(0, GROUPS, group_body, 0, unroll=False)
        pltpu.sync_copy(obuf, out_hbm.at[pl.ds(base, CHUNK)])
        return carry

    lax.fori_loop(0, N_CHUNKS, chunk_body, 0, unroll=False)


_sc_scorer = functools.partial(
    pl.kernel,
    mesh=plsc.VectorSubcoreMesh(core_axis_name="c", subcore_axis_name="s"),
    out_type=jax.ShapeDtypeStruct((N_EDGES,), jnp.float32),
    scratch_types=[
        pltpu.VMEM((CHUNK,), jnp.int32),
        pltpu.VMEM((CHUNK,), jnp.int32),
        pltpu.VMEM((CHUNK, HIDDEN), jnp.float32),
        pltpu.VMEM((CHUNK, HIDDEN), jnp.float32),
        pltpu.VMEM((HIDDEN,), jnp.float32),
        pltpu.VMEM((L,), jnp.float32),
        pltpu.VMEM((CHUNK,), jnp.float32),
        pltpu.SemaphoreType.DMA,
    ],
)(_sc_body)


def kernel(x, edge_index, W1, b1, W2, b2):
    z = _compute_z(x, W1, b1)
    ei = edge_index.astype(jnp.int32)
    ridx = ei[0]
    cidx = ei[1]
    w2 = W2[:, 0]
    b2v = jnp.broadcast_to(b2, (L,))
    return _sc_scorer(z, ridx, cidx, w2, b2v)


# speedup vs baseline: 6.4036x; 2.1341x over previous
import functools

import jax
import jax.numpy as jnp
from jax import lax
from jax.experimental import pallas as pl
from jax.experimental.pallas import tpu as pltpu
from jax.experimental.pallas import tpu_sc as plsc

NODE_DIM = 128
HIDDEN = 128
N_NODES = 10000
N_EDGES = 320000

NC = 2
NS = 16
NW = NC * NS
L = 16

EDGES_PER_W = N_EDGES // NW
CHUNK = 128
GROUPS = CHUNK // L
N_PAIRS = EDGES_PER_W // (2 * CHUNK)
TAIL = EDGES_PER_W - 2 * CHUNK * N_PAIRS
NVJ = HIDDEN // L


def _z_body(x_ref, w1_ref, b1_ref, z_ref):
    z_ref[...] = (
        jnp.dot(x_ref[...], w1_ref[...], preferred_element_type=jnp.float32)
        + b1_ref[...]
    ) * 0.5


def _compute_z(x, W1, b1):
    return pl.pallas_call(
        _z_body,
        out_shape=jax.ShapeDtypeStruct((N_NODES, HIDDEN), jnp.float32),
    )(x, W1, b1.reshape(1, HIDDEN))


_BITREV = (0, 8, 4, 12, 2, 10, 6, 14, 1, 9, 5, 13, 3, 11, 7, 15)

_GDN = lax.GatherDimensionNumbers(
    offset_dims=(), collapsed_slice_dims=(0,), start_index_map=(0,))


def _lane_perm(a, idx):
    return lax.gather(a, idx[:, None], _GDN, (1,),
                      mode=lax.GatherScatterMode.PROMISE_IN_BOUNDS)


def _hsum16(vregs, lane):
    cur = list(vregs)
    for w in (16, 8, 4, 2):
        swap = lane ^ (w // 2)
        low = (lane % w) < (w // 2)
        nxt = []
        for p in range(0, len(cur), 2):
            a, b = cur[p], cur[p + 1]
            pa = _lane_perm(a, swap)
            pb = _lane_perm(b, swap)
            nxt.append(jnp.where(low, a, pb) + jnp.where(low, pa, b))
        cur = nxt
    return cur[0]


def _sc_body(z_hbm, ridx_hbm, cidx_hbm, w2_hbm, b2_hbm, out_hbm,
             ridx_a, cidx_a, ridx_b, cidx_b,
             rrows_a, crows_a, rrows_b, crows_b,
             ridx_t, cidx_t, rrows_t, crows_t,
             w2_v, b2_v, obuf_a, obuf_b, obuf_t,
             sem_a, sem_b, sem_t):
    wid = lax.axis_index("s") * NC + lax.axis_index("c")
    base_w = wid * EDGES_PER_W

    pltpu.sync_copy(w2_hbm, w2_v)
    pltpu.sync_copy(b2_hbm, b2_v)
    w2r = [w2_v[pl.ds(j * L, L)] for j in range(NVJ)]
    b2r = b2_v[...]
    lane = lax.iota(jnp.int32, L)

    def start(k, ridx_v, cidx_v, rrows, crows, sem, n):
        base = base_w + k
        pltpu.sync_copy(ridx_hbm.at[pl.ds(base, n)], ridx_v)
        pltpu.sync_copy(cidx_hbm.at[pl.ds(base, n)], cidx_v)
        pltpu.async_copy(z_hbm.at[ridx_v], rrows, sem)
        pltpu.async_copy(z_hbm.at[cidx_v], crows, sem)

    def wait(ridx_v, cidx_v, rrows, crows, sem):
        pltpu.make_async_copy(z_hbm.at[ridx_v], rrows, sem).wait()
        pltpu.make_async_copy(z_hbm.at[cidx_v], crows, sem).wait()

    def group(rrows, crows, obuf, g):
        gbase = g * L
        accs = []
        for i in range(L):
            e = gbase + _BITREV[i]
            acc = jnp.zeros((L,), jnp.float32)
            for j in range(NVJ):
                rv = rrows[e, pl.ds(j * L, L)]
                cv = crows[e, pl.ds(j * L, L)]
                acc = acc + jnp.maximum(rv + cv, 0.0) * w2r[j]
            accs.append(acc)
        obuf[pl.ds(gbase, L)] = _hsum16(accs, lane) + b2r

    def compute_store(k, rrows, crows, obuf, n):
        def gbody(g, c):
            group(rrows, crows, obuf, g)
            return c
        lax.fori_loop(0, n // L, gbody, 0, unroll=False)
        pltpu.sync_copy(obuf, out_hbm.at[pl.ds(base_w + k, n)])

    start(0, ridx_a, cidx_a, rrows_a, crows_a, sem_a, CHUNK)
    start(CHUNK, ridx_b, cidx_b, rrows_b, crows_b, sem_b, CHUNK)

    def pair_body(t, carry):
        ka = 2 * t * CHUNK
        wait(ridx_a, cidx_a, rrows_a, crows_a, sem_a)
        compute_store(ka, rrows_a, crows_a, obuf_a, CHUNK)

        @pl.when(t < N_PAIRS - 1)
        def _():
            start(ka + 2 * CHUNK, ridx_a, cidx_a, rrows_a, crows_a, sem_a,
                  CHUNK)

        wait(ridx_b, cidx_b, rrows_b, crows_b, sem_b)
        compute_store(ka + CHUNK, rrows_b, crows_b, obuf_b, CHUNK)

        @pl.when(t < N_PAIRS - 1)
        def _():
            start(ka + 3 * CHUNK, ridx_b, cidx_b, rrows_b, crows_b, sem_b,
                  CHUNK)

        return carry

    lax.fori_loop(0, N_PAIRS, pair_body, 0, unroll=False)

    kt = 2 * CHUNK * N_PAIRS
    start(kt, ridx_t, cidx_t, rrows_t, crows_t, sem_t, TAIL)
    wait(ridx_t, cidx_t, rrows_t, crows_t, sem_t)
    group(rrows_t, crows_t, obuf_t, 0)
    pltpu.sync_copy(obuf_t, out_hbm.at[pl.ds(base_w + kt, TAIL)])


_sc_scorer = functools.partial(
    pl.kernel,
    mesh=plsc.VectorSubcoreMesh(core_axis_name="c", subcore_axis_name="s"),
    out_type=jax.ShapeDtypeStruct((N_EDGES,), jnp.float32),
    scratch_types=[
        pltpu.VMEM((CHUNK,), jnp.int32),
        pltpu.VMEM((CHUNK,), jnp.int32),
        pltpu.VMEM((CHUNK,), jnp.int32),
        pltpu.VMEM((CHUNK,), jnp.int32),
        pltpu.VMEM((CHUNK, HIDDEN), jnp.float32),
        pltpu.VMEM((CHUNK, HIDDEN), jnp.float32),
        pltpu.VMEM((CHUNK, HIDDEN), jnp.float32),
        pltpu.VMEM((CHUNK, HIDDEN), jnp.float32),
        pltpu.VMEM((TAIL,), jnp.int32),
        pltpu.VMEM((TAIL,), jnp.int32),
        pltpu.VMEM((TAIL, HIDDEN), jnp.float32),
        pltpu.VMEM((TAIL, HIDDEN), jnp.float32),
        pltpu.VMEM((HIDDEN,), jnp.float32),
        pltpu.VMEM((L,), jnp.float32),
        pltpu.VMEM((CHUNK,), jnp.float32),
        pltpu.VMEM((CHUNK,), jnp.float32),
        pltpu.VMEM((TAIL,), jnp.float32),
        pltpu.SemaphoreType.DMA,
        pltpu.SemaphoreType.DMA,
        pltpu.SemaphoreType.DMA,
    ],
)(_sc_body)


def kernel(x, edge_index, W1, b1, W2, b2):
    z = _compute_z(x, W1, b1)
    ei = edge_index.astype(jnp.int32)
    ridx = ei[0]
    cidx = ei[1]
    w2 = W2[:, 0]
    b2v = jnp.broadcast_to(b2, (L,))
    return _sc_scorer(z, ridx, cidx, w2, b2v)


# speedup vs baseline: 6.9765x; 1.0895x over previous
import functools

import jax
import jax.numpy as jnp
import numpy as np
from jax import lax
from jax.experimental import pallas as pl
from jax.experimental.pallas import tpu as pltpu
from jax.experimental.pallas import tpu_sc as plsc

NODE_DIM = 128
HIDDEN = 128
N_NODES = 10000
N_EDGES = 320000

NC = 2
NS = 16
NW = NC * NS
L = 16

EDGES_PER_W = N_EDGES // NW
CHUNK = 128
N_PAIRS = EDGES_PER_W // (2 * CHUNK)
TAIL = EDGES_PER_W - 2 * CHUNK * N_PAIRS
NBJ = HIDDEN // (2 * L)
PK = HIDDEN // 2


def _z_body(x_ref, w1_ref, b1_ref, z_ref):
    z_ref[...] = (
        (jnp.dot(x_ref[...], w1_ref[...], preferred_element_type=jnp.float32)
         + b1_ref[...]) * 0.5
    ).astype(jnp.bfloat16)


def _compute_z(x, W1, b1):
    return pl.pallas_call(
        _z_body,
        out_shape=jax.ShapeDtypeStruct((N_NODES, HIDDEN), jnp.bfloat16),
    )(x, W1, b1.reshape(1, HIDDEN))


_BITREV = (0, 8, 4, 12, 2, 10, 6, 14, 1, 9, 5, 13, 3, 11, 7, 15)

_GDN = lax.GatherDimensionNumbers(
    offset_dims=(), collapsed_slice_dims=(0,), start_index_map=(0,))


def _lane_perm(a, idx):
    return lax.gather(a, idx[:, None], _GDN, (1,),
                      mode=lax.GatherScatterMode.PROMISE_IN_BOUNDS)


def _hsum16(vregs, lane):
    cur = list(vregs)
    for w in (16, 8, 4, 2):
        swap = lane ^ (w // 2)
        low = (lane % w) < (w // 2)
        nxt = []
        for p in range(0, len(cur), 2):
            a, b = cur[p], cur[p + 1]
            pa = _lane_perm(a, swap)
            pb = _lane_perm(b, swap)
            nxt.append(jnp.where(low, a, pb) + jnp.where(low, pa, b))
        cur = nxt
    return cur[0]


def _sc_body(z_hbm, ridx_hbm, cidx_hbm, w2_hbm, b2_hbm, out_hbm,
             ridx_a, cidx_a, ridx_b, cidx_b,
             rrows_a, crows_a, rrows_b, crows_b,
             ridx_t, cidx_t, rrows_t, crows_t,
             w2_v, b2_v, obuf_a, obuf_b, obuf_t,
             sem_a, sem_b, sem_t):
    wid = lax.axis_index("s") * NC + lax.axis_index("c")
    base_w = wid * EDGES_PER_W

    pltpu.sync_copy(w2_hbm, w2_v)
    pltpu.sync_copy(b2_hbm, b2_v)
    w2e = [w2_v[pl.ds(j * 2 * L, L)] for j in range(NBJ)]
    w2o = [w2_v[pl.ds(j * 2 * L + L, L)] for j in range(NBJ)]
    b2r = b2_v[...]
    lane = lax.iota(jnp.int32, L)

    def start(k, ridx_v, cidx_v, rrows, crows, sem, n):
        base = base_w + k
        pltpu.sync_copy(ridx_hbm.at[pl.ds(base, n)], ridx_v)
        pltpu.sync_copy(cidx_hbm.at[pl.ds(base, n)], cidx_v)
        pltpu.async_copy(z_hbm.at[ridx_v], rrows, sem)
        pltpu.async_copy(z_hbm.at[cidx_v], crows, sem)

    def wait(ridx_v, cidx_v, rrows, crows, sem):
        pltpu.make_async_copy(z_hbm.at[ridx_v], rrows, sem).wait()
        pltpu.make_async_copy(z_hbm.at[cidx_v], crows, sem).wait()

    def group(rrows, crows, obuf, g):
        gbase = g * L
        accs = []
        for i in range(L):
            e = gbase + _BITREV[i]
            acc = jnp.zeros((L,), jnp.float32)
            for j in range(NBJ):
                rv = plsc.bitcast(rrows[e, pl.ds(j * L, L)], jnp.bfloat16)
                cv = plsc.bitcast(crows[e, pl.ds(j * L, L)], jnp.bfloat16)
                h = jnp.maximum(rv + cv, jnp.bfloat16(0))
                he, ho = plsc.unpack(h, format=plsc.PackFormat.INTERLEAVED)
                acc = acc + he * w2e[j] + ho * w2o[j]
            accs.append(acc)
        obuf[pl.ds(gbase, L)] = _hsum16(accs, lane) + b2r

    def compute_store(k, rrows, crows, obuf, n):
        def gbody(g, c):
            group(rrows, crows, obuf, g)
            return c
        lax.fori_loop(0, n // L, gbody, 0, unroll=False)
        pltpu.sync_copy(obuf, out_hbm.at[pl.ds(base_w + k, n)])

    start(0, ridx_a, cidx_a, rrows_a, crows_a, sem_a, CHUNK)
    start(CHUNK, ridx_b, cidx_b, rrows_b, crows_b, sem_b, CHUNK)

    def pair_body(t, carry):
        ka = 2 * t * CHUNK
        wait(ridx_a, cidx_a, rrows_a, crows_a, sem_a)
        compute_store(ka, rrows_a, crows_a, obuf_a, CHUNK)

        @pl.when(t < N_PAIRS - 1)
        def _():
            start(ka + 2 * CHUNK, ridx_a, cidx_a, rrows_a, crows_a, sem_a,
                  CHUNK)

        wait(ridx_b, cidx_b, rrows_b, crows_b, sem_b)
        compute_store(ka + CHUNK, rrows_b, crows_b, obuf_b, CHUNK)

        @pl.when(t < N_PAIRS - 1)
        def _():
            start(ka + 3 * CHUNK, ridx_b, cidx_b, rrows_b, crows_b, sem_b,
                  CHUNK)

        return carry

    lax.fori_loop(0, N_PAIRS, pair_body, 0, unroll=False)

    kt = 2 * CHUNK * N_PAIRS
    start(kt, ridx_t, cidx_t, rrows_t, crows_t, sem_t, TAIL)
    wait(ridx_t, cidx_t, rrows_t, crows_t, sem_t)
    group(rrows_t, crows_t, obuf_t, 0)
    pltpu.sync_copy(obuf_t, out_hbm.at[pl.ds(base_w + kt, TAIL)])


_sc_scorer = functools.partial(
    pl.kernel,
    mesh=plsc.VectorSubcoreMesh(core_axis_name="c", subcore_axis_name="s"),
    out_type=jax.ShapeDtypeStruct((N_EDGES,), jnp.float32),
    compiler_params=pltpu.CompilerParams(
        needs_layout_passes=False, use_tc_tiling_on_sc=False),
    scratch_types=[
        pltpu.VMEM((CHUNK,), jnp.int32),
        pltpu.VMEM((CHUNK,), jnp.int32),
        pltpu.VMEM((CHUNK,), jnp.int32),
        pltpu.VMEM((CHUNK,), jnp.int32),
        pltpu.VMEM((CHUNK, PK), jnp.int32),
        pltpu.VMEM((CHUNK, PK), jnp.int32),
        pltpu.VMEM((CHUNK, PK), jnp.int32),
        pltpu.VMEM((CHUNK, PK), jnp.int32),
        pltpu.VMEM((TAIL,), jnp.int32),
        pltpu.VMEM((TAIL,), jnp.int32),
        pltpu.VMEM((TAIL, PK), jnp.int32),
        pltpu.VMEM((TAIL, PK), jnp.int32),
        pltpu.VMEM((HIDDEN,), jnp.float32),
        pltpu.VMEM((L,), jnp.float32),
        pltpu.VMEM((CHUNK,), jnp.float32),
        pltpu.VMEM((CHUNK,), jnp.float32),
        pltpu.VMEM((TAIL,), jnp.float32),
        pltpu.SemaphoreType.DMA,
        pltpu.SemaphoreType.DMA,
        pltpu.SemaphoreType.DMA,
    ],
)(_sc_body)

_W2_PERM = np.concatenate(
    [np.concatenate([np.arange(32 * j, 32 * (j + 1), 2),
                     np.arange(32 * j + 1, 32 * (j + 1), 2)])
     for j in range(NBJ)])


def kernel(x, edge_index, W1, b1, W2, b2):
    z = _compute_z(x, W1, b1)
    z = lax.bitcast_convert_type(z.reshape(N_NODES, PK, 2), jnp.int32)
    ei = edge_index.astype(jnp.int32)
    ridx = ei[0]
    cidx = ei[1]
    w2 = W2[:, 0][_W2_PERM]
    b2v = jnp.broadcast_to(b2, (L,))
    return _sc_scorer(z, ridx, cidx, w2, b2v)


# speedup vs baseline: 10.1698x; 1.4577x over previous
import functools

import jax
import jax.numpy as jnp
import numpy as np
from jax import lax
from jax.experimental import pallas as pl
from jax.experimental.pallas import tpu as pltpu
from jax.experimental.pallas import tpu_sc as plsc

NODE_DIM = 128
HIDDEN = 128
N_NODES = 10000
N_EDGES = 320000

NC = 2
NS = 16
NW = NC * NS
L = 16

EDGES_PER_W = N_EDGES // NW
CHUNK = 128
NSETS = 4
N_CHUNKS = EDGES_PER_W // CHUNK
N_RING = N_CHUNKS // NSETS
TAIL = EDGES_PER_W - N_CHUNKS * CHUNK
NBJ = HIDDEN // (2 * L)
PK = HIDDEN // 2


def _z_body(x_ref, w1_ref, b1_ref, z_ref):
    z_ref[...] = (
        (jnp.dot(x_ref[...], w1_ref[...], preferred_element_type=jnp.float32)
         + b1_ref[...]) * 0.5
    ).astype(jnp.bfloat16)


def _compute_z(x, W1, b1):
    return pl.pallas_call(
        _z_body,
        out_shape=jax.ShapeDtypeStruct((N_NODES, HIDDEN), jnp.bfloat16),
    )(x, W1, b1.reshape(1, HIDDEN))


_BITREV = (0, 8, 4, 12, 2, 10, 6, 14, 1, 9, 5, 13, 3, 11, 7, 15)

_GDN = lax.GatherDimensionNumbers(
    offset_dims=(), collapsed_slice_dims=(0,), start_index_map=(0,))


def _lane_perm(a, idx):
    return lax.gather(a, idx[:, None], _GDN, (1,),
                      mode=lax.GatherScatterMode.PROMISE_IN_BOUNDS)


def _hsum16(vregs, lane):
    cur = list(vregs)
    for w in (16, 8, 4, 2):
        swap = lane ^ (w // 2)
        low = (lane % w) < (w // 2)
        nxt = []
        for p in range(0, len(cur), 2):
            a, b = cur[p], cur[p + 1]
            pa = _lane_perm(a, swap)
            pb = _lane_perm(b, swap)
            nxt.append(jnp.where(low, a, pb) + jnp.where(low, pa, b))
        cur = nxt
    return cur[0]


def _sc_body(z_hbm, ridx_hbm, cidx_hbm, w2_hbm, b2_hbm, out_hbm,
             ridx_all, cidx_all,
             rrows0, crows0, rrows1, crows1,
             rrows2, crows2, rrows3, crows3,
             obuf0, obuf1, obuf2, obuf3,
             rrows_t, crows_t, obuf_t, w2_v, b2_v,
             gsem0, gsem1, gsem2, gsem3,
             ssem0, ssem1, ssem2, ssem3, tsem):
    wid = lax.axis_index("s") * NC + lax.axis_index("c")
    base_w = wid * EDGES_PER_W

    rrows = (rrows0, rrows1, rrows2, rrows3)
    crows = (crows0, crows1, crows2, crows3)
    obufs = (obuf0, obuf1, obuf2, obuf3)
    gsems = (gsem0, gsem1, gsem2, gsem3)
    ssems = (ssem0, ssem1, ssem2, ssem3)

    pltpu.sync_copy(ridx_hbm.at[pl.ds(base_w, EDGES_PER_W)], ridx_all)
    pltpu.sync_copy(cidx_hbm.at[pl.ds(base_w, EDGES_PER_W)], cidx_all)
    pltpu.sync_copy(w2_hbm, w2_v)
    pltpu.sync_copy(b2_hbm, b2_v)
    w2e = [w2_v[pl.ds(j * 2 * L, L)] for j in range(NBJ)]
    w2o = [w2_v[pl.ds(j * 2 * L + L, L)] for j in range(NBJ)]
    b2r = b2_v[...]
    lane = lax.iota(jnp.int32, L)

    def start(k, s):
        ri = ridx_all.at[pl.ds(k * CHUNK, CHUNK)]
        ci = cidx_all.at[pl.ds(k * CHUNK, CHUNK)]
        pltpu.async_copy(z_hbm.at[ri], rrows[s], gsems[s])
        pltpu.async_copy(z_hbm.at[ci], crows[s], gsems[s])

    def wait(k, s):
        ri = ridx_all.at[pl.ds(k * CHUNK, CHUNK)]
        ci = cidx_all.at[pl.ds(k * CHUNK, CHUNK)]
        pltpu.make_async_copy(z_hbm.at[ri], rrows[s], gsems[s]).wait()
        pltpu.make_async_copy(z_hbm.at[ci], crows[s], gsems[s]).wait()

    def group(rr, cr, obuf, g):
        gbase = g * L
        accs = []
        for i in range(L):
            e = gbase + _BITREV[i]
            acc = jnp.zeros((L,), jnp.float32)
            for j in range(NBJ):
                rv = plsc.bitcast(rr[e, pl.ds(j * L, L)], jnp.bfloat16)
                cv = plsc.bitcast(cr[e, pl.ds(j * L, L)], jnp.bfloat16)
                h = jnp.maximum(rv + cv, jnp.bfloat16(0))
                he, ho = plsc.unpack(h, format=plsc.PackFormat.INTERLEAVED)
                acc = acc + he * w2e[j] + ho * w2o[j]
            accs.append(acc)
        obuf[pl.ds(gbase, L)] = _hsum16(accs, lane) + b2r

    def compute_store(k, s, t):
        @pl.when(t > 0)
        def _():
            pltpu.make_async_copy(
                obufs[s], out_hbm.at[pl.ds(base_w, CHUNK)], ssems[s]).wait()

        def gbody(g, c):
            group(rrows[s], crows[s], obufs[s], g)
            return c
        lax.fori_loop(0, CHUNK // L, gbody, 0, unroll=False)
        pltpu.async_copy(
            obufs[s], out_hbm.at[pl.ds(base_w + k * CHUNK, CHUNK)], ssems[s])

    for s in range(NSETS):
        start(s, s)

    def ring_body(t, carry):
        for s in range(NSETS):
            k = t * NSETS + s
            wait(k, s)
            compute_store(k, s, t)

            @pl.when(k + NSETS < N_CHUNKS)
            def _():
                start(k + NSETS, s)
        return carry

    lax.fori_loop(0, N_RING, ring_body, 0, unroll=False)

    for s in range(N_CHUNKS - N_RING * NSETS):
        k = N_RING * NSETS + s
        wait(k, s)
        compute_store(k, s, N_RING)

    kt = N_CHUNKS * CHUNK
    ri = ridx_all.at[pl.ds(kt, TAIL)]
    ci = cidx_all.at[pl.ds(kt, TAIL)]
    pltpu.async_copy(z_hbm.at[ri], rrows_t, tsem)
    pltpu.async_copy(z_hbm.at[ci], crows_t, tsem)
    pltpu.make_async_copy(z_hbm.at[ri], rrows_t, tsem).wait()
    pltpu.make_async_copy(z_hbm.at[ci], crows_t, tsem).wait()
    group(rrows_t, crows_t, obuf_t, 0)
    pltpu.sync_copy(obuf_t, out_hbm.at[pl.ds(base_w + kt, TAIL)])

    for s in range(NSETS):
        k_last = N_RING * NSETS + s if s < N_CHUNKS - N_RING * NSETS else 0
        pltpu.make_async_copy(
            obufs[s], out_hbm.at[pl.ds(base_w + k_last * CHUNK, CHUNK)],
            ssems[s]).wait()


_sc_scorer = functools.partial(
    pl.kernel,
    mesh=plsc.VectorSubcoreMesh(core_axis_name="c", subcore_axis_name="s"),
    out_type=jax.ShapeDtypeStruct((N_EDGES,), jnp.float32),
    compiler_params=pltpu.CompilerParams(
        needs_layout_passes=False, use_tc_tiling_on_sc=False),
    scratch_types=[
        pltpu.VMEM((EDGES_PER_W,), jnp.int32),
        pltpu.VMEM((EDGES_PER_W,), jnp.int32),
        pltpu.VMEM((CHUNK, PK), jnp.int32),
        pltpu.VMEM((CHUNK, PK), jnp.int32),
        pltpu.VMEM((CHUNK, PK), jnp.int32),
        pltpu.VMEM((CHUNK, PK), jnp.int32),
        pltpu.VMEM((CHUNK, PK), jnp.int32),
        pltpu.VMEM((CHUNK, PK), jnp.int32),
        pltpu.VMEM((CHUNK, PK), jnp.int32),
        pltpu.VMEM((CHUNK, PK), jnp.int32),
        pltpu.VMEM((CHUNK,), jnp.float32),
        pltpu.VMEM((CHUNK,), jnp.float32),
        pltpu.VMEM((CHUNK,), jnp.float32),
        pltpu.VMEM((CHUNK,), jnp.float32),
        pltpu.VMEM((TAIL, PK), jnp.int32),
        pltpu.VMEM((TAIL, PK), jnp.int32),
        pltpu.VMEM((TAIL,), jnp.float32),
        pltpu.VMEM((HIDDEN,), jnp.float32),
        pltpu.VMEM((L,), jnp.float32),
        pltpu.SemaphoreType.DMA,
        pltpu.SemaphoreType.DMA,
        pltpu.SemaphoreType.DMA,
        pltpu.SemaphoreType.DMA,
        pltpu.SemaphoreType.DMA,
        pltpu.SemaphoreType.DMA,
        pltpu.SemaphoreType.DMA,
        pltpu.SemaphoreType.DMA,
        pltpu.SemaphoreType.DMA,
    ],
)(_sc_body)

_W2_PERM = np.concatenate(
    [np.concatenate([np.arange(32 * j, 32 * (j + 1), 2),
                     np.arange(32 * j + 1, 32 * (j + 1), 2)])
     for j in range(NBJ)])


def kernel(x, edge_index, W1, b1, W2, b2):
    z = _compute_z(x, W1, b1)
    z = lax.bitcast_convert_type(z.reshape(N_NODES, PK, 2), jnp.int32)
    ei = edge_index.astype(jnp.int32)
    ridx = ei[0]
    cidx = ei[1]
    w2 = W2[:, 0][_W2_PERM]
    b2v = jnp.broadcast_to(b2, (L,))
    return _sc_scorer(z, ridx, cidx, w2, b2v)
